# baseline (speedup 1.0000x reference)
import functools

import jax
import jax.numpy as jnp
from jax import lax
from jax.experimental import pallas as pl
from jax.experimental.pallas import tpu as pltpu
from jax.experimental.pallas import tpu_sc as plsc

N = 10000
E = 320000
F = 128
H = 128
G = 64
C = 2

NPAD = 10240
DUMMY = NPAD - 1
EPAD = 327680
NCHUNKS = EPAD // 128
NTILES = 32
CPT = NCHUNKS // NTILES
RPT = NPAD // 16
GRP = 2
IB = 16


def _sc_mesh():
    return plsc.VectorSubcoreMesh(
        core_axis_name="c", subcore_axis_name="s", num_cores=2, num_subcores=16
    )


def _make_degree(width):
    @functools.partial(
        pl.kernel,
        out_type=jax.ShapeDtypeStruct((2, NPAD, width), jnp.float32),
        mesh=_sc_mesh(),
        scratch_types=[
            pltpu.VMEM((CPT, 128), jnp.int32),
            pltpu.VMEM((128, width), jnp.float32),
            pltpu.VMEM_SHARED((NPAD, width), jnp.float32),
        ],
    )
    def degree(dsts, ones_hbm, zeros, out, dst_v, ones_v, accum):
        c = lax.axis_index("c")
        s = lax.axis_index("s")
        wid = s * 2 + c
        pltpu.sync_copy(dsts.at[pl.ds(wid * CPT, CPT)], dst_v)
        pltpu.sync_copy(ones_hbm, ones_v)
        pltpu.sync_copy(zeros, accum.at[pl.ds(s * RPT, RPT)])
        plsc.subcore_barrier()

        def step(j, carry):
            pltpu.sync_copy(ones_v, accum.at[dst_v.at[j]], add=True)
            return carry

        lax.fori_loop(0, CPT, step, 0)
        plsc.subcore_barrier()
        pltpu.sync_copy(
            accum.at[pl.ds(s * RPT, RPT)], out.at[c, pl.ds(s * RPT, RPT)]
        )

    return degree


def _make_prop(width):
    mesh = _sc_mesh()

    @functools.partial(
        pl.kernel,
        out_type=jax.ShapeDtypeStruct((2, NPAD, width), jnp.float32),
        mesh=mesh,
        scratch_types=[
            pltpu.VMEM((IB, 128), jnp.int32),
            pltpu.VMEM((IB, 128), jnp.int32),
            pltpu.VMEM((128, width), jnp.float32),
            pltpu.VMEM((128, width), jnp.float32),
            pltpu.VMEM_SHARED((NPAD, width), jnp.float32),
            pltpu.SemaphoreType.DMA,
            pltpu.SemaphoreType.DMA,
        ],
    )
    def prop(table, srcs, dsts, zeros, out, src_v, dst_v, buf0, buf1, accum, sem0, sem1):
        c = lax.axis_index("c")
        s = lax.axis_index("s")
        wid = s * 2 + c
        pltpu.sync_copy(zeros, accum.at[pl.ds(s * RPT, RPT)])
        plsc.subcore_barrier()

        bufs = (buf0, buf1)
        sems = (sem0, sem1)

        def block(ib, carry):
            base = wid * CPT + ib * IB
            pltpu.sync_copy(srcs.at[pl.ds(base, IB)], src_v)
            pltpu.sync_copy(dsts.at[pl.ds(base, IB)], dst_v)

            def outer(jo, carry2):
                descs = []
                for b in range(GRP):
                    j = jo * GRP + b
                    descs.append(
                        pltpu.async_copy(table.at[src_v.at[j]], bufs[b], sems[b])
                    )
                for b in range(GRP):
                    j = jo * GRP + b
                    descs[b].wait()
                    pltpu.sync_copy(bufs[b], accum.at[dst_v.at[j]], add=True)
                return carry2

            lax.fori_loop(0, IB // GRP, outer, 0)
            return carry

        lax.fori_loop(0, CPT // IB, block, 0)
        plsc.subcore_barrier()
        pltpu.sync_copy(
            accum.at[pl.ds(s * RPT, RPT)], out.at[c, pl.ds(s * RPT, RPT)]
        )

    return prop


NCH2 = 2688
CPT2 = NCH2 // 16
NBODY = CPT2 // 3
DSTBLK = 24
IDXPAD = 16
EPAD2 = NCH2 * 128
HALF = NPAD // 2
HROW = HALF + 128
LDUMMY = HROW - 1
RPT2 = HROW // 16


def _make_prop_split():
    @functools.partial(
        pl.kernel,
        out_type=jax.ShapeDtypeStruct((2, HROW, 128), jnp.float32),
        mesh=_sc_mesh(),
        scratch_types=[
            pltpu.VMEM((CPT2 + 8, 128), jnp.int32),
            pltpu.VMEM((DSTBLK, 128), jnp.int32),
            pltpu.VMEM((128, 128), jnp.float32),
            pltpu.VMEM((128, 128), jnp.float32),
            pltpu.VMEM((128, 128), jnp.float32),
            pltpu.VMEM_SHARED((HROW, 128), jnp.float32),
            pltpu.SemaphoreType.DMA,
            pltpu.SemaphoreType.DMA,
            pltpu.SemaphoreType.DMA,
            pltpu.SemaphoreType.DMA,
            pltpu.SemaphoreType.DMA,
            pltpu.SemaphoreType.DMA,
        ],
    )
    def prop(table, srcs, dsts2, zeros, out,
             src_all, dst_v, b0, b1, b2, accum,
             g0, g1, g2, s0, s1, s2):
        c = lax.axis_index("c")
        s = lax.axis_index("s")
        base_g = s * CPT2
        bufs = (b0, b1, b2)
        gsem = (g0, g1, g2)
        ssem = (s0, s1, s2)

        pltpu.sync_copy(srcs.at[pl.ds(base_g, CPT2 + 8)], src_all)
        pltpu.sync_copy(zeros.at[pl.ds(0, RPT2)], accum.at[pl.ds(s * RPT2, RPT2)])
        plsc.subcore_barrier()

        def gath(st, row):
            return pltpu.async_copy(table.at[src_all.at[row]], bufs[st], gsem[st])

        def gath_wait(st):
            pltpu.make_async_copy(table.at[src_all.at[0]], bufs[st], gsem[st]).wait()

        def scat(st, drow):
            return pltpu.async_copy(
                bufs[st], accum.at[dst_v.at[drow]], ssem[st], add=True
            )

        def scat_wait(st):
            pltpu.make_async_copy(bufs[st], accum.at[dst_v.at[0]], ssem[st]).wait()

        pltpu.sync_copy(dsts2.at[pl.ds(c * NCH2 + base_g, DSTBLK)], dst_v)
        pltpu.sync_copy(zeros.at[pl.ds(0, 128)], b2)
        scat(2, 0)
        gath(0, 0)
        gath(1, 1)

        def body(i, carry):
            bl = i * 3
            dr = lax.rem(i, 8) * 3
            scat_wait(2)

            @pl.when(lax.rem(i, 8) == 0)
            def _stage_dst():
                pltpu.sync_copy(
                    dsts2.at[pl.ds(c * NCH2 + base_g + (i // 8) * DSTBLK, DSTBLK)],
                    dst_v,
                )

            gath_wait(0)
            scat(0, dr + 0)
            gath(2, bl + 2)
            gath_wait(1)
            scat(1, dr + 1)
            scat_wait(0)
            gath(0, bl + 3)
            gath_wait(2)
            scat(2, dr + 2)
            scat_wait(1)
            gath(1, bl + 4)
            return carry

        lax.fori_loop(0, NBODY, body, 0)

        scat_wait(2)
        gath_wait(0)
        gath_wait(1)
        plsc.subcore_barrier()
        pltpu.sync_copy(
            accum.at[pl.ds(s * RPT2, RPT2)], out.at[c, pl.ds(s * RPT2, RPT2)]
        )

    return prop


_degree128 = _make_degree(128)
_prop128 = _make_prop(128)
_prop_split = _make_prop_split()

SRCPAD = 16 * CPT2 + 8


def _prep_body(degp0, degp1, x_ref, norm_ref, hn_ref):
    deg = degp0[:, 0:1] + degp1[:, 0:1] + 1.0
    norm = lax.rsqrt(deg)
    norm_ref[...] = norm
    hn_ref[...] = x_ref[...] * norm


def _prep(degp0, degp1, x_p):
    return pl.pallas_call(
        _prep_body,
        out_shape=(
            jax.ShapeDtypeStruct((NPAD, 1), jnp.float32),
            jax.ShapeDtypeStruct((NPAD, H), jnp.float32),
        ),
    )(degp0, degp1, x_p)


def _layer_body(p0, p1, hn, norm, W, b, out):
    for half, ph in ((0, p0), (1, p1)):
        rows = pl.ds(half * HALF, HALF)
        agg = (ph[...] + hn[rows, :]) * norm[rows, :]
        h = jnp.maximum(
            jnp.dot(agg, W[...], preferred_element_type=jnp.float32) + b[...], 0.0
        )
        out[rows, :] = h * norm[rows, :]


def _layer(p0, p1, hn, norm, W, b):
    return pl.pallas_call(
        _layer_body,
        out_shape=jax.ShapeDtypeStruct((NPAD, H), jnp.float32),
    )(p0, p1, hn, norm, W, b)


NSCAN = 14


def _final_body(
    p0, p1, hn, norm, W, b, batch_ref, same_ref, end_ref, Wm1, Wm2, bm,
    out, h_scr, pa, pb,
):
    for half, ph in ((0, p0), (1, p1)):
        rows = pl.ds(half * HALF, HALF)
        agg = (ph[...] + hn[rows, :]) * norm[rows, :]
        h_scr[rows, :] = jnp.maximum(
            jnp.dot(agg, W[...], preferred_element_type=jnp.float32) + b[...], 0.0
        )

    bufs = (pa, pb)
    src = h_scr
    for k in range(NSCAN):
        dst = bufs[k % 2]
        d = 1 << k
        L = NPAD - d
        dst[pl.ds(0, d), :] = src[pl.ds(0, d), :]
        shifted = src[pl.ds(0, L), :]
        cur = src[pl.ds(d, L), :]
        same = same_ref[pl.ds(d, L), k:k + 1]
        dst[pl.ds(d, L), :] = jnp.maximum(
            cur, jnp.where(same > 0.0, shifted, -1e30)
        )
        src = dst
    pref = src

    ones = jnp.ones((128, 128), jnp.float32)

    def chunk(ci, carry):
        ms, mx, cnt = carry
        hc = h_scr[pl.ds(ci * 128, 128), :]
        pc = pref[pl.ds(ci * 128, 128), :]
        bc = batch_ref[0:1, pl.ds(ci * 128, 128)]
        ec = end_ref[0:1, pl.ds(ci * 128, 128)]
        ids = lax.broadcasted_iota(jnp.int32, (G, 128), 0)
        eqf = (ids == bc).astype(jnp.float32)
        ms = ms + jnp.dot(eqf, hc, preferred_element_type=jnp.float32)
        cnt = cnt + jnp.dot(eqf, ones, preferred_element_type=jnp.float32)
        mx = mx + jnp.dot(eqf * ec, pc, preferred_element_type=jnp.float32)
        return ms, mx, cnt

    init = (
        jnp.zeros((G, H), jnp.float32),
        jnp.zeros((G, H), jnp.float32),
        jnp.zeros((G, H), jnp.float32),
    )
    ms, mx, cnt = lax.fori_loop(0, NPAD // 128, chunk, init)
    meanp = ms / jnp.maximum(cnt, 1.0)
    logits = (
        jnp.dot(meanp, Wm1[...], preferred_element_type=jnp.float32)
        + jnp.dot(mx, Wm2[...], preferred_element_type=jnp.float32)
        + bm[...]
    )
    out[...] = jax.nn.sigmoid(logits)


def _final(p0, p1, hn, norm, W, b, batch_p, same_m, end_m, Wm1, Wm2, bm):
    return pl.pallas_call(
        _final_body,
        out_shape=jax.ShapeDtypeStruct((G, C), jnp.float32),
        scratch_shapes=[
            pltpu.VMEM((NPAD, H), jnp.float32),
            pltpu.VMEM((NPAD, H), jnp.float32),
            pltpu.VMEM((NPAD, H), jnp.float32),
        ],
    )(p0, p1, hn, norm, W, b, batch_p, same_m, end_m, Wm1, Wm2, bm)


def kernel(x, edge_index, batch, W0, b0, W1, b1, W2, b2, Wm, bm):
    src = edge_index[0]
    dst = edge_index[1]
    pad_e = EPAD - E
    dst_p = jnp.concatenate(
        [dst, jnp.full((pad_e,), DUMMY, jnp.int32)]
    ).reshape(NCHUNKS, 128)
    src_p = jnp.concatenate(
        [src, jnp.zeros((pad_e,), jnp.int32)]
    ).reshape(NCHUNKS, 128)
    x_p = jnp.pad(x, ((0, NPAD - N), (0, 0)))
    batch_pad = jnp.pad(batch, (0, NPAD - N), constant_values=2**30)
    batch_p = batch_pad.reshape(1, NPAD)
    same_cols = [
        jnp.concatenate(
            [jnp.zeros((1 << k,), jnp.bool_), batch_pad[1 << k:] == batch_pad[:-(1 << k)]]
        )
        for k in range(NSCAN)
    ]
    same_m = jnp.stack(
        same_cols + [jnp.zeros((NPAD,), jnp.bool_)] * (16 - NSCAN), axis=1
    ).astype(jnp.float32)
    end_m = jnp.concatenate(
        [batch_pad[:-1] != batch_pad[1:], jnp.ones((1,), jnp.bool_)]
    ).astype(jnp.float32).reshape(1, NPAD)
    zeros_w = jnp.zeros((RPT, H), jnp.float32)
    ones_tab = jnp.ones((128, 128), jnp.float32)

    pad_e2 = EPAD2 - E
    src_p2 = jnp.concatenate(
        [src, jnp.zeros((pad_e2 + 8 * 128,), jnp.int32)]
    ).reshape(SRCPAD, 128)
    dst_pad2 = jnp.concatenate([dst, jnp.full((pad_e2,), 2 * NPAD, jnp.int32)])
    dst_locals = []
    for c in range(2):
        rel = dst_pad2 - c * HALF
        dst_locals.append(
            jnp.where((rel >= 0) & (rel < HALF), rel, LDUMMY).astype(jnp.int32)
        )
    dst_p2 = jnp.concatenate(dst_locals).reshape(2 * NCH2, 128)
    zeros_h = jnp.zeros((RPT2, H), jnp.float32)

    degp = _degree128(dst_p, ones_tab, zeros_w)
    norm, hn = _prep(degp[0], degp[1], x_p)
    p = _prop_split(hn, src_p2, dst_p2, zeros_h)
    hn = _layer(p[0, :HALF], p[1, :HALF], hn, norm, W0, b0.reshape(1, H))
    p = _prop_split(hn, src_p2, dst_p2, zeros_h)
    hn = _layer(p[0, :HALF], p[1, :HALF], hn, norm, W1, b1.reshape(1, H))
    p = _prop_split(hn, src_p2, dst_p2, zeros_h)
    return _final(
        p[0, :HALF], p[1, :HALF], hn, norm, W2, b2.reshape(1, H),
        batch_p, same_m, end_m, Wm[:H], Wm[H:], bm.reshape(1, C),
    )


# speedup vs baseline: 3.6023x; 3.6023x over previous
import functools

import jax
import jax.numpy as jnp
from jax import lax
from jax.experimental import pallas as pl
from jax.experimental.pallas import tpu as pltpu
from jax.experimental.pallas import tpu_sc as plsc

N = 10000
E = 320000
F = 128
H = 128
G = 64
C = 2

NPAD = 10240
DUMMY = NPAD - 1
EPAD = 327680
NCHUNKS = EPAD // 128
NTILES = 32
CPT = NCHUNKS // NTILES
RPT = NPAD // 16
GRP = 2
IB = 16


def _sc_mesh():
    return plsc.VectorSubcoreMesh(
        core_axis_name="c", subcore_axis_name="s", num_cores=2, num_subcores=16
    )


def _make_degree(width):
    @functools.partial(
        pl.kernel,
        out_type=jax.ShapeDtypeStruct((2, NPAD, width), jnp.float32),
        mesh=_sc_mesh(),
        scratch_types=[
            pltpu.VMEM((CPT, 128), jnp.int32),
            pltpu.VMEM((128, width), jnp.float32),
            pltpu.VMEM_SHARED((NPAD, width), jnp.float32),
        ],
    )
    def degree(dsts, ones_hbm, zeros, out, dst_v, ones_v, accum):
        c = lax.axis_index("c")
        s = lax.axis_index("s")
        wid = s * 2 + c
        pltpu.sync_copy(dsts.at[pl.ds(wid * CPT, CPT)], dst_v)
        pltpu.sync_copy(ones_hbm, ones_v)
        pltpu.sync_copy(zeros, accum.at[pl.ds(s * RPT, RPT)])
        plsc.subcore_barrier()

        def step(j, carry):
            pltpu.sync_copy(ones_v, accum.at[dst_v.at[j]], add=True)
            return carry

        lax.fori_loop(0, CPT, step, 0)
        plsc.subcore_barrier()
        pltpu.sync_copy(
            accum.at[pl.ds(s * RPT, RPT)], out.at[c, pl.ds(s * RPT, RPT)]
        )

    return degree


def _make_prop(width):
    mesh = _sc_mesh()

    @functools.partial(
        pl.kernel,
        out_type=jax.ShapeDtypeStruct((2, NPAD, width), jnp.float32),
        mesh=mesh,
        scratch_types=[
            pltpu.VMEM((IB, 128), jnp.int32),
            pltpu.VMEM((IB, 128), jnp.int32),
            pltpu.VMEM((128, width), jnp.float32),
            pltpu.VMEM((128, width), jnp.float32),
            pltpu.VMEM_SHARED((NPAD, width), jnp.float32),
            pltpu.SemaphoreType.DMA,
            pltpu.SemaphoreType.DMA,
        ],
    )
    def prop(table, srcs, dsts, zeros, out, src_v, dst_v, buf0, buf1, accum, sem0, sem1):
        c = lax.axis_index("c")
        s = lax.axis_index("s")
        wid = s * 2 + c
        pltpu.sync_copy(zeros, accum.at[pl.ds(s * RPT, RPT)])
        plsc.subcore_barrier()

        bufs = (buf0, buf1)
        sems = (sem0, sem1)

        def block(ib, carry):
            base = wid * CPT + ib * IB
            pltpu.sync_copy(srcs.at[pl.ds(base, IB)], src_v)
            pltpu.sync_copy(dsts.at[pl.ds(base, IB)], dst_v)

            def outer(jo, carry2):
                descs = []
                for b in range(GRP):
                    j = jo * GRP + b
                    descs.append(
                        pltpu.async_copy(table.at[src_v.at[j]], bufs[b], sems[b])
                    )
                for b in range(GRP):
                    j = jo * GRP + b
                    descs[b].wait()
                    pltpu.sync_copy(bufs[b], accum.at[dst_v.at[j]], add=True)
                return carry2

            lax.fori_loop(0, IB // GRP, outer, 0)
            return carry

        lax.fori_loop(0, CPT // IB, block, 0)
        plsc.subcore_barrier()
        pltpu.sync_copy(
            accum.at[pl.ds(s * RPT, RPT)], out.at[c, pl.ds(s * RPT, RPT)]
        )

    return prop


NCH2 = 2688
CPT2 = NCH2 // 16
NBODY = CPT2 // 3
DSTBLK = 24
IDXPAD = 16
EPAD2 = NCH2 * 128
HALF = NPAD // 2
HROW = HALF + 128
LDUMMY = HROW - 1
RPT2 = HROW // 16


def _make_prop_split():
    @functools.partial(
        pl.kernel,
        out_type=jax.ShapeDtypeStruct((2, HROW, 128), jnp.float32),
        mesh=_sc_mesh(),
        scratch_types=[
            pltpu.VMEM((CPT2 + 8, 128), jnp.int32),
            pltpu.VMEM((DSTBLK, 128), jnp.int32),
            pltpu.VMEM((128, 128), jnp.float32),
            pltpu.VMEM((128, 128), jnp.float32),
            pltpu.VMEM((128, 128), jnp.float32),
            pltpu.VMEM_SHARED((HROW, 128), jnp.float32),
            pltpu.SemaphoreType.DMA,
            pltpu.SemaphoreType.DMA,
            pltpu.SemaphoreType.DMA,
            pltpu.SemaphoreType.DMA,
            pltpu.SemaphoreType.DMA,
            pltpu.SemaphoreType.DMA,
        ],
    )
    def prop(table, srcs, dsts2, zeros, out,
             src_all, dst_v, b0, b1, b2, accum,
             g0, g1, g2, s0, s1, s2):
        c = lax.axis_index("c")
        s = lax.axis_index("s")
        base_g = s * CPT2
        bufs = (b0, b1, b2)
        gsem = (g0, g1, g2)
        ssem = (s0, s1, s2)

        pltpu.sync_copy(srcs.at[pl.ds(base_g, CPT2 + 8)], src_all)
        pltpu.sync_copy(zeros, accum.at[pl.ds(s * RPT2, RPT2)])
        plsc.subcore_barrier()

        def gath(st, row):
            return pltpu.async_copy(table.at[src_all.at[row]], bufs[st], gsem[st])

        def gath_wait(st):
            pltpu.make_async_copy(table.at[src_all.at[0]], bufs[st], gsem[st]).wait()

        def scat(st, drow):
            return pltpu.async_copy(
                bufs[st], accum.at[dst_v.at[drow]], ssem[st], add=True
            )

        def scat_wait(st):
            pltpu.make_async_copy(bufs[st], accum.at[dst_v.at[0]], ssem[st]).wait()

        pltpu.sync_copy(dsts2.at[pl.ds(c * NCH2 + base_g, DSTBLK)], dst_v)
        pltpu.sync_copy(zeros.at[pl.ds(0, 128)], b2)
        scat(2, 0)
        gath(0, 0)
        gath(1, 1)

        def body(i, carry):
            bl = i * 3
            dr = lax.rem(i, 8) * 3
            scat_wait(2)

            @pl.when(lax.rem(i, 8) == 0)
            def _stage_dst():
                pltpu.sync_copy(
                    dsts2.at[pl.ds(c * NCH2 + base_g + (i // 8) * DSTBLK, DSTBLK)],
                    dst_v,
                )

            gath_wait(0)
            scat(0, dr + 0)
            gath(2, bl + 2)
            gath_wait(1)
            scat(1, dr + 1)
            scat_wait(0)
            gath(0, bl + 3)
            gath_wait(2)
            scat(2, dr + 2)
            scat_wait(1)
            gath(1, bl + 4)
            return carry

        lax.fori_loop(0, NBODY, body, 0)

        scat_wait(2)
        gath_wait(0)
        gath_wait(1)
        plsc.subcore_barrier()
        pltpu.sync_copy(
            accum.at[pl.ds(s * RPT2, RPT2)], out.at[c, pl.ds(s * RPT2, RPT2)]
        )

    return prop


_degree128 = _make_degree(128)
_prop128 = _make_prop(128)


def _prep_body(degp0, degp1, x_ref, norm_ref, hn_ref):
    deg = degp0[:, 0:1] + degp1[:, 0:1] + 1.0
    norm = lax.rsqrt(deg)
    norm_ref[...] = norm
    hn_ref[...] = x_ref[...] * norm


def _prep(degp0, degp1, x_p):
    return pl.pallas_call(
        _prep_body,
        out_shape=(
            jax.ShapeDtypeStruct((NPAD, 1), jnp.float32),
            jax.ShapeDtypeStruct((NPAD, H), jnp.float32),
        ),
    )(degp0, degp1, x_p)


def _layer_body(p0, p1, hn, norm, W, b, out):
    agg = (p0[...] + p1[...] + hn[...]) * norm[...]
    h = jnp.maximum(
        jnp.dot(agg, W[...], preferred_element_type=jnp.float32) + b[...], 0.0
    )
    out[...] = h * norm[...]


def _layer(p0, p1, hn, norm, W, b):
    return pl.pallas_call(
        _layer_body,
        out_shape=jax.ShapeDtypeStruct((NPAD, H), jnp.float32),
    )(p0, p1, hn, norm, W, b)


NSCAN = 14


def _final_body(
    p0, p1, hn, norm, W, b, batch_ref, same_ref, end_ref, Wm1, Wm2, bm,
    out, h_scr, pa, pb,
):
    agg = (p0[...] + p1[...] + hn[...]) * norm[...]
    h_scr[...] = jnp.maximum(
        jnp.dot(agg, W[...], preferred_element_type=jnp.float32) + b[...], 0.0
    )

    bufs = (pa, pb)
    src = h_scr
    for k in range(NSCAN):
        dst = bufs[k % 2]
        d = 1 << k
        L = NPAD - d
        dst[pl.ds(0, d), :] = src[pl.ds(0, d), :]
        shifted = src[pl.ds(0, L), :]
        cur = src[pl.ds(d, L), :]
        same = same_ref[pl.ds(d, L), k:k + 1]
        dst[pl.ds(d, L), :] = jnp.maximum(
            cur, jnp.where(same > 0.0, shifted, -1e30)
        )
        src = dst
    pref = src

    ones = jnp.ones((128, 128), jnp.float32)

    def chunk(ci, carry):
        ms, mx, cnt = carry
        hc = h_scr[pl.ds(ci * 128, 128), :]
        pc = pref[pl.ds(ci * 128, 128), :]
        bc = batch_ref[0:1, pl.ds(ci * 128, 128)]
        ec = end_ref[0:1, pl.ds(ci * 128, 128)]
        ids = lax.broadcasted_iota(jnp.int32, (G, 128), 0)
        eqf = (ids == bc).astype(jnp.float32)
        ms = ms + jnp.dot(eqf, hc, preferred_element_type=jnp.float32)
        cnt = cnt + jnp.dot(eqf, ones, preferred_element_type=jnp.float32)
        mx = mx + jnp.dot(eqf * ec, pc, preferred_element_type=jnp.float32)
        return ms, mx, cnt

    init = (
        jnp.zeros((G, H), jnp.float32),
        jnp.zeros((G, H), jnp.float32),
        jnp.zeros((G, H), jnp.float32),
    )
    ms, mx, cnt = lax.fori_loop(0, NPAD // 128, chunk, init)
    meanp = ms / jnp.maximum(cnt, 1.0)
    logits = (
        jnp.dot(meanp, Wm1[...], preferred_element_type=jnp.float32)
        + jnp.dot(mx, Wm2[...], preferred_element_type=jnp.float32)
        + bm[...]
    )
    out[...] = jax.nn.sigmoid(logits)


def _final(p0, p1, hn, norm, W, b, batch_p, same_m, end_m, Wm1, Wm2, bm):
    return pl.pallas_call(
        _final_body,
        out_shape=jax.ShapeDtypeStruct((G, C), jnp.float32),
        scratch_shapes=[
            pltpu.VMEM((NPAD, H), jnp.float32),
            pltpu.VMEM((NPAD, H), jnp.float32),
            pltpu.VMEM((NPAD, H), jnp.float32),
        ],
    )(p0, p1, hn, norm, W, b, batch_p, same_m, end_m, Wm1, Wm2, bm)


def kernel(x, edge_index, batch, W0, b0, W1, b1, W2, b2, Wm, bm):
    src = edge_index[0]
    dst = edge_index[1]
    pad_e = EPAD - E
    dst_p = jnp.concatenate(
        [dst, jnp.full((pad_e,), DUMMY, jnp.int32)]
    ).reshape(NCHUNKS, 128)
    src_p = jnp.concatenate(
        [src, jnp.zeros((pad_e,), jnp.int32)]
    ).reshape(NCHUNKS, 128)
    x_p = jnp.pad(x, ((0, NPAD - N), (0, 0)))
    batch_pad = jnp.pad(batch, (0, NPAD - N), constant_values=2**30)
    batch_p = batch_pad.reshape(1, NPAD)
    same_cols = [
        jnp.concatenate(
            [jnp.zeros((1 << k,), jnp.bool_), batch_pad[1 << k:] == batch_pad[:-(1 << k)]]
        )
        for k in range(NSCAN)
    ]
    same_m = jnp.stack(
        same_cols + [jnp.zeros((NPAD,), jnp.bool_)] * (16 - NSCAN), axis=1
    ).astype(jnp.float32)
    end_m = jnp.concatenate(
        [batch_pad[:-1] != batch_pad[1:], jnp.ones((1,), jnp.bool_)]
    ).astype(jnp.float32).reshape(1, NPAD)
    zeros_w = jnp.zeros((RPT, H), jnp.float32)
    ones_tab = jnp.ones((128, 128), jnp.float32)

    degp = _degree128(dst_p, ones_tab, zeros_w)
    norm, hn = _prep(degp[0], degp[1], x_p)
    p = _prop128(hn, src_p, dst_p, zeros_w)
    hn = _layer(p[0], p[1], hn, norm, W0, b0.reshape(1, H))
    p = _prop128(hn, src_p, dst_p, zeros_w)
    hn = _layer(p[0], p[1], hn, norm, W1, b1.reshape(1, H))
    p = _prop128(hn, src_p, dst_p, zeros_w)
    return _final(
        p[0], p[1], hn, norm, W2, b2.reshape(1, H),
        batch_p, same_m, end_m, Wm[:H], Wm[H:], bm.reshape(1, C),
    )


# speedup vs baseline: 3.7920x; 1.0526x over previous
import functools

import jax
import jax.numpy as jnp
from jax import lax
from jax.experimental import pallas as pl
from jax.experimental.pallas import tpu as pltpu
from jax.experimental.pallas import tpu_sc as plsc

N = 10000
E = 320000
F = 128
H = 128
G = 64
C = 2

NPAD = 10240
DUMMY = NPAD - 1
EPAD = 327680
NCHUNKS = EPAD // 128
NTILES = 32
CPT = NCHUNKS // NTILES
RPT = NPAD // 16
GRP = 2
IB = 16


def _sc_mesh():
    return plsc.VectorSubcoreMesh(
        core_axis_name="c", subcore_axis_name="s", num_cores=2, num_subcores=16
    )


def _make_degree(width):
    @functools.partial(
        pl.kernel,
        out_type=jax.ShapeDtypeStruct((2, NPAD, width), jnp.float32),
        mesh=_sc_mesh(),
        scratch_types=[
            pltpu.VMEM((CPT, 128), jnp.int32),
            pltpu.VMEM((128, width), jnp.float32),
            pltpu.VMEM_SHARED((NPAD, width), jnp.float32),
            pltpu.SemaphoreType.DMA,
            pltpu.SemaphoreType.DMA,
            pltpu.SemaphoreType.DMA,
            pltpu.SemaphoreType.DMA,
        ],
    )
    def degree(dsts, ones_hbm, zeros, out, dst_v, ones_v, accum, d0, d1, d2, d3):
        c = lax.axis_index("c")
        s = lax.axis_index("s")
        wid = s * 2 + c
        sems = (d0, d1, d2, d3)
        pltpu.sync_copy(dsts.at[pl.ds(wid * CPT, CPT)], dst_v)
        pltpu.sync_copy(ones_hbm, ones_v)
        pltpu.sync_copy(zeros, accum.at[pl.ds(s * RPT, RPT)])
        plsc.subcore_barrier()

        def scat(j, b):
            return pltpu.async_copy(ones_v, accum.at[dst_v.at[j]], sems[b], add=True)

        def swait(b):
            pltpu.make_async_copy(ones_v, accum.at[dst_v.at[0]], sems[b]).wait()

        for b in range(4):
            scat(b, b)

        def step(jo, carry):
            for b in range(4):
                swait(b)
                scat(jo * 4 + b, b)
            return carry

        lax.fori_loop(1, CPT // 4, step, 0)
        for b in range(4):
            swait(b)
        plsc.subcore_barrier()
        pltpu.sync_copy(
            accum.at[pl.ds(s * RPT, RPT)], out.at[c, pl.ds(s * RPT, RPT)]
        )

    return degree


def _make_prop(width):
    mesh = _sc_mesh()

    @functools.partial(
        pl.kernel,
        out_type=jax.ShapeDtypeStruct((2, NPAD, width), jnp.float32),
        mesh=mesh,
        scratch_types=[
            pltpu.VMEM((IB, 128), jnp.int32),
            pltpu.VMEM((IB, 128), jnp.int32),
            pltpu.VMEM((128, width), jnp.float32),
            pltpu.VMEM((128, width), jnp.float32),
            pltpu.VMEM_SHARED((NPAD, width), jnp.float32),
            pltpu.SemaphoreType.DMA,
            pltpu.SemaphoreType.DMA,
            pltpu.SemaphoreType.DMA,
            pltpu.SemaphoreType.DMA,
        ],
    )
    def prop(table, srcs, dsts, zeros, out, src_v, dst_v, buf0, buf1, accum,
             g0, g1, s0, s1):
        c = lax.axis_index("c")
        s = lax.axis_index("s")
        wid = s * 2 + c
        pltpu.sync_copy(zeros, accum.at[pl.ds(s * RPT, RPT)])
        plsc.subcore_barrier()

        bufs = (buf0, buf1)
        gsems = (g0, g1)
        ssems = (s0, s1)

        def gath(j, b):
            return pltpu.async_copy(table.at[src_v.at[j]], bufs[b], gsems[b])

        def gwait(b):
            pltpu.make_async_copy(table.at[src_v.at[0]], bufs[b], gsems[b]).wait()

        def scat(j, b):
            return pltpu.async_copy(bufs[b], accum.at[dst_v.at[j]], ssems[b], add=True)

        def swait(b):
            pltpu.make_async_copy(bufs[b], accum.at[dst_v.at[0]], ssems[b]).wait()

        def block(ib, carry):
            base = wid * CPT + ib * IB
            pltpu.sync_copy(srcs.at[pl.ds(base, IB)], src_v)
            pltpu.sync_copy(dsts.at[pl.ds(base, IB)], dst_v)

            for b in range(GRP):
                gath(b, b)

            def outer(jo, carry2):
                for b in range(GRP):
                    gwait(b)
                    scat(jo * GRP + b, b)
                for b in range(GRP):
                    swait(b)
                    gath((jo + 1) * GRP + b, b)
                return carry2

            lax.fori_loop(0, IB // GRP - 1, outer, 0)
            last = IB - GRP
            for b in range(GRP):
                gwait(b)
                scat(last + b, b)
            for b in range(GRP):
                swait(b)
            return carry

        lax.fori_loop(0, CPT // IB, block, 0)
        plsc.subcore_barrier()
        pltpu.sync_copy(
            accum.at[pl.ds(s * RPT, RPT)], out.at[c, pl.ds(s * RPT, RPT)]
        )

    return prop


NCH2 = 2688
CPT2 = NCH2 // 16
NBODY = CPT2 // 3
DSTBLK = 24
IDXPAD = 16
EPAD2 = NCH2 * 128
HALF = NPAD // 2
HROW = HALF + 128
LDUMMY = HROW - 1
RPT2 = HROW // 16


def _make_prop_split():
    @functools.partial(
        pl.kernel,
        out_type=jax.ShapeDtypeStruct((2, HROW, 128), jnp.float32),
        mesh=_sc_mesh(),
        scratch_types=[
            pltpu.VMEM((CPT2 + 8, 128), jnp.int32),
            pltpu.VMEM((DSTBLK, 128), jnp.int32),
            pltpu.VMEM((128, 128), jnp.float32),
            pltpu.VMEM((128, 128), jnp.float32),
            pltpu.VMEM((128, 128), jnp.float32),
            pltpu.VMEM_SHARED((HROW, 128), jnp.float32),
            pltpu.SemaphoreType.DMA,
            pltpu.SemaphoreType.DMA,
            pltpu.SemaphoreType.DMA,
            pltpu.SemaphoreType.DMA,
            pltpu.SemaphoreType.DMA,
            pltpu.SemaphoreType.DMA,
        ],
    )
    def prop(table, srcs, dsts2, zeros, out,
             src_all, dst_v, b0, b1, b2, accum,
             g0, g1, g2, s0, s1, s2):
        c = lax.axis_index("c")
        s = lax.axis_index("s")
        base_g = s * CPT2
        bufs = (b0, b1, b2)
        gsem = (g0, g1, g2)
        ssem = (s0, s1, s2)

        pltpu.sync_copy(srcs.at[pl.ds(base_g, CPT2 + 8)], src_all)
        pltpu.sync_copy(zeros, accum.at[pl.ds(s * RPT2, RPT2)])
        plsc.subcore_barrier()

        def gath(st, row):
            return pltpu.async_copy(table.at[src_all.at[row]], bufs[st], gsem[st])

        def gath_wait(st):
            pltpu.make_async_copy(table.at[src_all.at[0]], bufs[st], gsem[st]).wait()

        def scat(st, drow):
            return pltpu.async_copy(
                bufs[st], accum.at[dst_v.at[drow]], ssem[st], add=True
            )

        def scat_wait(st):
            pltpu.make_async_copy(bufs[st], accum.at[dst_v.at[0]], ssem[st]).wait()

        pltpu.sync_copy(dsts2.at[pl.ds(c * NCH2 + base_g, DSTBLK)], dst_v)
        pltpu.sync_copy(zeros.at[pl.ds(0, 128)], b2)
        scat(2, 0)
        gath(0, 0)
        gath(1, 1)

        def body(i, carry):
            bl = i * 3
            dr = lax.rem(i, 8) * 3
            scat_wait(2)

            @pl.when(lax.rem(i, 8) == 0)
            def _stage_dst():
                pltpu.sync_copy(
                    dsts2.at[pl.ds(c * NCH2 + base_g + (i // 8) * DSTBLK, DSTBLK)],
                    dst_v,
                )

            gath_wait(0)
            scat(0, dr + 0)
            gath(2, bl + 2)
            gath_wait(1)
            scat(1, dr + 1)
            scat_wait(0)
            gath(0, bl + 3)
            gath_wait(2)
            scat(2, dr + 2)
            scat_wait(1)
            gath(1, bl + 4)
            return carry

        lax.fori_loop(0, NBODY, body, 0)

        scat_wait(2)
        gath_wait(0)
        gath_wait(1)
        plsc.subcore_barrier()
        pltpu.sync_copy(
            accum.at[pl.ds(s * RPT2, RPT2)], out.at[c, pl.ds(s * RPT2, RPT2)]
        )

    return prop


_degree128 = _make_degree(128)
_prop128 = _make_prop(128)


def _prep_body(degp0, degp1, x_ref, norm_ref, hn_ref):
    deg = degp0[:, 0:1] + degp1[:, 0:1] + 1.0
    norm = lax.rsqrt(deg)
    norm_ref[...] = norm
    hn_ref[...] = x_ref[...] * norm


def _prep(degp0, degp1, x_p):
    return pl.pallas_call(
        _prep_body,
        out_shape=(
            jax.ShapeDtypeStruct((NPAD, 1), jnp.float32),
            jax.ShapeDtypeStruct((NPAD, H), jnp.float32),
        ),
    )(degp0, degp1, x_p)


def _layer_body(p0, p1, hn, norm, W, b, out):
    agg = (p0[...] + p1[...] + hn[...]) * norm[...]
    h = jnp.maximum(
        jnp.dot(agg, W[...], preferred_element_type=jnp.float32) + b[...], 0.0
    )
    out[...] = h * norm[...]


def _layer(p0, p1, hn, norm, W, b):
    return pl.pallas_call(
        _layer_body,
        out_shape=jax.ShapeDtypeStruct((NPAD, H), jnp.float32),
    )(p0, p1, hn, norm, W, b)


NSCAN = 14


def _final_body(
    p0, p1, hn, norm, W, b, batch_ref, same_ref, end_ref, Wm1, Wm2, bm,
    out, h_scr, pa, pb,
):
    agg = (p0[...] + p1[...] + hn[...]) * norm[...]
    h_scr[...] = jnp.maximum(
        jnp.dot(agg, W[...], preferred_element_type=jnp.float32) + b[...], 0.0
    )

    bufs = (pa, pb)
    src = h_scr
    for k in range(NSCAN):
        dst = bufs[k % 2]
        d = 1 << k
        L = NPAD - d
        dst[pl.ds(0, d), :] = src[pl.ds(0, d), :]
        shifted = src[pl.ds(0, L), :]
        cur = src[pl.ds(d, L), :]
        same = same_ref[pl.ds(d, L), k:k + 1]
        dst[pl.ds(d, L), :] = jnp.maximum(
            cur, jnp.where(same > 0.0, shifted, -1e30)
        )
        src = dst
    pref = src

    ones = jnp.ones((128, 128), jnp.float32)

    def chunk(ci, carry):
        ms, mx, cnt = carry
        hc = h_scr[pl.ds(ci * 128, 128), :]
        pc = pref[pl.ds(ci * 128, 128), :]
        bc = batch_ref[0:1, pl.ds(ci * 128, 128)]
        ec = end_ref[0:1, pl.ds(ci * 128, 128)]
        ids = lax.broadcasted_iota(jnp.int32, (G, 128), 0)
        eqf = (ids == bc).astype(jnp.float32)
        ms = ms + jnp.dot(eqf, hc, preferred_element_type=jnp.float32)
        cnt = cnt + jnp.dot(eqf, ones, preferred_element_type=jnp.float32)
        mx = mx + jnp.dot(eqf * ec, pc, preferred_element_type=jnp.float32)
        return ms, mx, cnt

    init = (
        jnp.zeros((G, H), jnp.float32),
        jnp.zeros((G, H), jnp.float32),
        jnp.zeros((G, H), jnp.float32),
    )
    ms, mx, cnt = lax.fori_loop(0, NPAD // 128, chunk, init)
    meanp = ms / jnp.maximum(cnt, 1.0)
    logits = (
        jnp.dot(meanp, Wm1[...], preferred_element_type=jnp.float32)
        + jnp.dot(mx, Wm2[...], preferred_element_type=jnp.float32)
        + bm[...]
    )
    out[...] = jax.nn.sigmoid(logits)


def _final(p0, p1, hn, norm, W, b, batch_p, same_m, end_m, Wm1, Wm2, bm):
    return pl.pallas_call(
        _final_body,
        out_shape=jax.ShapeDtypeStruct((G, C), jnp.float32),
        scratch_shapes=[
            pltpu.VMEM((NPAD, H), jnp.float32),
            pltpu.VMEM((NPAD, H), jnp.float32),
            pltpu.VMEM((NPAD, H), jnp.float32),
        ],
    )(p0, p1, hn, norm, W, b, batch_p, same_m, end_m, Wm1, Wm2, bm)


def kernel(x, edge_index, batch, W0, b0, W1, b1, W2, b2, Wm, bm):
    src = edge_index[0]
    dst = edge_index[1]
    pad_e = EPAD - E
    dst_p = jnp.concatenate(
        [dst, jnp.full((pad_e,), DUMMY, jnp.int32)]
    ).reshape(NCHUNKS, 128)
    src_p = jnp.concatenate(
        [src, jnp.zeros((pad_e,), jnp.int32)]
    ).reshape(NCHUNKS, 128)
    x_p = jnp.pad(x, ((0, NPAD - N), (0, 0)))
    batch_pad = jnp.pad(batch, (0, NPAD - N), constant_values=2**30)
    batch_p = batch_pad.reshape(1, NPAD)
    same_cols = [
        jnp.concatenate(
            [jnp.zeros((1 << k,), jnp.bool_), batch_pad[1 << k:] == batch_pad[:-(1 << k)]]
        )
        for k in range(NSCAN)
    ]
    same_m = jnp.stack(
        same_cols + [jnp.zeros((NPAD,), jnp.bool_)] * (16 - NSCAN), axis=1
    ).astype(jnp.float32)
    end_m = jnp.concatenate(
        [batch_pad[:-1] != batch_pad[1:], jnp.ones((1,), jnp.bool_)]
    ).astype(jnp.float32).reshape(1, NPAD)
    zeros_w = jnp.zeros((RPT, H), jnp.float32)
    ones_tab = jnp.ones((128, 128), jnp.float32)

    degp = _degree128(dst_p, ones_tab, zeros_w)
    norm, hn = _prep(degp[0], degp[1], x_p)
    p = _prop128(hn, src_p, dst_p, zeros_w)
    hn = _layer(p[0], p[1], hn, norm, W0, b0.reshape(1, H))
    p = _prop128(hn, src_p, dst_p, zeros_w)
    hn = _layer(p[0], p[1], hn, norm, W1, b1.reshape(1, H))
    p = _prop128(hn, src_p, dst_p, zeros_w)
    return _final(
        p[0], p[1], hn, norm, W2, b2.reshape(1, H),
        batch_p, same_m, end_m, Wm[:H], Wm[H:], bm.reshape(1, C),
    )


# speedup vs baseline: 3.8968x; 1.0276x over previous
import functools

import jax
import jax.numpy as jnp
from jax import lax
from jax.experimental import pallas as pl
from jax.experimental.pallas import tpu as pltpu
from jax.experimental.pallas import tpu_sc as plsc

N = 10000
E = 320000
F = 128
H = 128
G = 64
C = 2

NPAD = 10240
DUMMY = NPAD - 1
EPAD = 327680
NCHUNKS = EPAD // 128
NTILES = 32
CPT = NCHUNKS // NTILES
RPT = NPAD // 16
GRP = 2
IB = 16


def _sc_mesh():
    return plsc.VectorSubcoreMesh(
        core_axis_name="c", subcore_axis_name="s", num_cores=2, num_subcores=16
    )


def _make_degree(width):
    @functools.partial(
        pl.kernel,
        out_type=jax.ShapeDtypeStruct((2, NPAD, width), jnp.float32),
        mesh=_sc_mesh(),
        scratch_types=[
            pltpu.VMEM((CPT, 128), jnp.int32),
            pltpu.VMEM((128, width), jnp.float32),
            pltpu.VMEM_SHARED((NPAD, width), jnp.float32),
            pltpu.SemaphoreType.DMA,
            pltpu.SemaphoreType.DMA,
            pltpu.SemaphoreType.DMA,
            pltpu.SemaphoreType.DMA,
        ],
    )
    def degree(dsts, ones_hbm, zeros, out, dst_v, ones_v, accum, d0, d1, d2, d3):
        c = lax.axis_index("c")
        s = lax.axis_index("s")
        wid = s * 2 + c
        sems = (d0, d1, d2, d3)
        pltpu.sync_copy(dsts.at[pl.ds(wid * CPT, CPT)], dst_v)
        pltpu.sync_copy(ones_hbm, ones_v)
        pltpu.sync_copy(zeros, accum.at[pl.ds(s * RPT, RPT)])
        plsc.subcore_barrier()

        def scat(j, b):
            return pltpu.async_copy(ones_v, accum.at[dst_v.at[j]], sems[b], add=True)

        def swait(b):
            pltpu.make_async_copy(ones_v, accum.at[dst_v.at[0]], sems[b]).wait()

        for b in range(4):
            scat(b, b)

        def step(jo, carry):
            for b in range(4):
                swait(b)
                scat(jo * 4 + b, b)
            return carry

        lax.fori_loop(1, CPT // 4, step, 0)
        for b in range(4):
            swait(b)
        plsc.subcore_barrier()
        pltpu.sync_copy(
            accum.at[pl.ds(s * RPT, RPT)], out.at[c, pl.ds(s * RPT, RPT)]
        )

    return degree


SUB = 64
NSUB = EPAD // SUB
SPT = NSUB // NTILES
IB2 = 32
GRP2 = 4


def _make_prop(width):
    mesh = _sc_mesh()

    @functools.partial(
        pl.kernel,
        out_type=jax.ShapeDtypeStruct((2, NPAD, width), jnp.float32),
        mesh=mesh,
        scratch_types=[
            pltpu.VMEM((IB2, SUB), jnp.int32),
            pltpu.VMEM((IB2, SUB), jnp.int32),
            pltpu.VMEM((SUB, width), jnp.float32),
            pltpu.VMEM((SUB, width), jnp.float32),
            pltpu.VMEM((SUB, width), jnp.float32),
            pltpu.VMEM((SUB, width), jnp.float32),
            pltpu.VMEM_SHARED((NPAD, width), jnp.float32),
            pltpu.SemaphoreType.DMA,
            pltpu.SemaphoreType.DMA,
            pltpu.SemaphoreType.DMA,
            pltpu.SemaphoreType.DMA,
            pltpu.SemaphoreType.DMA,
            pltpu.SemaphoreType.DMA,
            pltpu.SemaphoreType.DMA,
            pltpu.SemaphoreType.DMA,
        ],
    )
    def prop(table, srcs, dsts, zeros, out, src_v, dst_v, buf0, buf1, buf2, buf3,
             accum, g0, g1, g2, g3, s0, s1, s2, s3):
        c = lax.axis_index("c")
        s = lax.axis_index("s")
        wid = s * 2 + c
        pltpu.sync_copy(zeros, accum.at[pl.ds(s * RPT, RPT)])
        plsc.subcore_barrier()

        bufs = (buf0, buf1, buf2, buf3)
        gsems = (g0, g1, g2, g3)
        ssems = (s0, s1, s2, s3)

        def gath(j, b):
            return pltpu.async_copy(table.at[src_v.at[j]], bufs[b], gsems[b])

        def gwait(b):
            pltpu.make_async_copy(table.at[src_v.at[0]], bufs[b], gsems[b]).wait()

        def scat(j, b):
            return pltpu.async_copy(bufs[b], accum.at[dst_v.at[j]], ssems[b], add=True)

        def swait(b):
            pltpu.make_async_copy(bufs[b], accum.at[dst_v.at[0]], ssems[b]).wait()

        def block(ib, carry):
            base = wid * SPT + ib * IB2
            pltpu.sync_copy(srcs.at[pl.ds(base, IB2)], src_v)
            pltpu.sync_copy(dsts.at[pl.ds(base, IB2)], dst_v)

            for b in range(GRP2):
                gath(b, b)

            def outer(jo, carry2):
                for b in range(GRP2):
                    gwait(b)
                    scat(jo * GRP2 + b, b)
                for b in range(GRP2):
                    swait(b)
                    gath((jo + 1) * GRP2 + b, b)
                return carry2

            lax.fori_loop(0, IB2 // GRP2 - 1, outer, 0)
            last = IB2 - GRP2
            for b in range(GRP2):
                gwait(b)
                scat(last + b, b)
            for b in range(GRP2):
                swait(b)
            return carry

        lax.fori_loop(0, SPT // IB2, block, 0)
        plsc.subcore_barrier()
        pltpu.sync_copy(
            accum.at[pl.ds(s * RPT, RPT)], out.at[c, pl.ds(s * RPT, RPT)]
        )

    return prop


NCH2 = 2688
CPT2 = NCH2 // 16
NBODY = CPT2 // 3
DSTBLK = 24
IDXPAD = 16
EPAD2 = NCH2 * 128
HALF = NPAD // 2
HROW = HALF + 128
LDUMMY = HROW - 1
RPT2 = HROW // 16


def _make_prop_split():
    @functools.partial(
        pl.kernel,
        out_type=jax.ShapeDtypeStruct((2, HROW, 128), jnp.float32),
        mesh=_sc_mesh(),
        scratch_types=[
            pltpu.VMEM((CPT2 + 8, 128), jnp.int32),
            pltpu.VMEM((DSTBLK, 128), jnp.int32),
            pltpu.VMEM((128, 128), jnp.float32),
            pltpu.VMEM((128, 128), jnp.float32),
            pltpu.VMEM((128, 128), jnp.float32),
            pltpu.VMEM_SHARED((HROW, 128), jnp.float32),
            pltpu.SemaphoreType.DMA,
            pltpu.SemaphoreType.DMA,
            pltpu.SemaphoreType.DMA,
            pltpu.SemaphoreType.DMA,
            pltpu.SemaphoreType.DMA,
            pltpu.SemaphoreType.DMA,
        ],
    )
    def prop(table, srcs, dsts2, zeros, out,
             src_all, dst_v, b0, b1, b2, accum,
             g0, g1, g2, s0, s1, s2):
        c = lax.axis_index("c")
        s = lax.axis_index("s")
        base_g = s * CPT2
        bufs = (b0, b1, b2)
        gsem = (g0, g1, g2)
        ssem = (s0, s1, s2)

        pltpu.sync_copy(srcs.at[pl.ds(base_g, CPT2 + 8)], src_all)
        pltpu.sync_copy(zeros, accum.at[pl.ds(s * RPT2, RPT2)])
        plsc.subcore_barrier()

        def gath(st, row):
            return pltpu.async_copy(table.at[src_all.at[row]], bufs[st], gsem[st])

        def gath_wait(st):
            pltpu.make_async_copy(table.at[src_all.at[0]], bufs[st], gsem[st]).wait()

        def scat(st, drow):
            return pltpu.async_copy(
                bufs[st], accum.at[dst_v.at[drow]], ssem[st], add=True
            )

        def scat_wait(st):
            pltpu.make_async_copy(bufs[st], accum.at[dst_v.at[0]], ssem[st]).wait()

        pltpu.sync_copy(dsts2.at[pl.ds(c * NCH2 + base_g, DSTBLK)], dst_v)
        pltpu.sync_copy(zeros.at[pl.ds(0, 128)], b2)
        scat(2, 0)
        gath(0, 0)
        gath(1, 1)

        def body(i, carry):
            bl = i * 3
            dr = lax.rem(i, 8) * 3
            scat_wait(2)

            @pl.when(lax.rem(i, 8) == 0)
            def _stage_dst():
                pltpu.sync_copy(
                    dsts2.at[pl.ds(c * NCH2 + base_g + (i // 8) * DSTBLK, DSTBLK)],
                    dst_v,
                )

            gath_wait(0)
            scat(0, dr + 0)
            gath(2, bl + 2)
            gath_wait(1)
            scat(1, dr + 1)
            scat_wait(0)
            gath(0, bl + 3)
            gath_wait(2)
            scat(2, dr + 2)
            scat_wait(1)
            gath(1, bl + 4)
            return carry

        lax.fori_loop(0, NBODY, body, 0)

        scat_wait(2)
        gath_wait(0)
        gath_wait(1)
        plsc.subcore_barrier()
        pltpu.sync_copy(
            accum.at[pl.ds(s * RPT2, RPT2)], out.at[c, pl.ds(s * RPT2, RPT2)]
        )

    return prop


_degree128 = _make_degree(128)
_prop128 = _make_prop(128)


def _prep_body(degp0, degp1, x_ref, norm_ref, hn_ref):
    deg = degp0[:, 0:1] + degp1[:, 0:1] + 1.0
    norm = lax.rsqrt(deg)
    norm_ref[...] = norm
    hn_ref[...] = x_ref[...] * norm


def _prep(degp0, degp1, x_p):
    return pl.pallas_call(
        _prep_body,
        out_shape=(
            jax.ShapeDtypeStruct((NPAD, 1), jnp.float32),
            jax.ShapeDtypeStruct((NPAD, H), jnp.float32),
        ),
    )(degp0, degp1, x_p)


def _layer_body(p0, p1, hn, norm, W, b, out):
    agg = (p0[...] + p1[...] + hn[...]) * norm[...]
    h = jnp.maximum(
        jnp.dot(agg, W[...], preferred_element_type=jnp.float32) + b[...], 0.0
    )
    out[...] = h * norm[...]


def _layer(p0, p1, hn, norm, W, b):
    return pl.pallas_call(
        _layer_body,
        out_shape=jax.ShapeDtypeStruct((NPAD, H), jnp.float32),
    )(p0, p1, hn, norm, W, b)


NSCAN = 14


def _final_body(
    p0, p1, hn, norm, W, b, batch_ref, same_ref, end_ref, Wm1, Wm2, bm,
    out, h_scr, pa, pb,
):
    agg = (p0[...] + p1[...] + hn[...]) * norm[...]
    h_scr[...] = jnp.maximum(
        jnp.dot(agg, W[...], preferred_element_type=jnp.float32) + b[...], 0.0
    )

    bufs = (pa, pb)
    src = h_scr
    for k in range(NSCAN):
        dst = bufs[k % 2]
        d = 1 << k
        L = NPAD - d
        dst[pl.ds(0, d), :] = src[pl.ds(0, d), :]
        shifted = src[pl.ds(0, L), :]
        cur = src[pl.ds(d, L), :]
        same = same_ref[pl.ds(d, L), k:k + 1]
        dst[pl.ds(d, L), :] = jnp.maximum(
            cur, jnp.where(same > 0.0, shifted, -1e30)
        )
        src = dst
    pref = src

    ones = jnp.ones((128, 128), jnp.float32)

    def chunk(ci, carry):
        ms, mx, cnt = carry
        hc = h_scr[pl.ds(ci * 128, 128), :]
        pc = pref[pl.ds(ci * 128, 128), :]
        bc = batch_ref[0:1, pl.ds(ci * 128, 128)]
        ec = end_ref[0:1, pl.ds(ci * 128, 128)]
        ids = lax.broadcasted_iota(jnp.int32, (G, 128), 0)
        eqf = (ids == bc).astype(jnp.float32)
        ms = ms + jnp.dot(eqf, hc, preferred_element_type=jnp.float32)
        cnt = cnt + jnp.dot(eqf, ones, preferred_element_type=jnp.float32)
        mx = mx + jnp.dot(eqf * ec, pc, preferred_element_type=jnp.float32)
        return ms, mx, cnt

    init = (
        jnp.zeros((G, H), jnp.float32),
        jnp.zeros((G, H), jnp.float32),
        jnp.zeros((G, H), jnp.float32),
    )
    ms, mx, cnt = lax.fori_loop(0, NPAD // 128, chunk, init)
    meanp = ms / jnp.maximum(cnt, 1.0)
    logits = (
        jnp.dot(meanp, Wm1[...], preferred_element_type=jnp.float32)
        + jnp.dot(mx, Wm2[...], preferred_element_type=jnp.float32)
        + bm[...]
    )
    out[...] = jax.nn.sigmoid(logits)


def _final(p0, p1, hn, norm, W, b, batch_p, same_m, end_m, Wm1, Wm2, bm):
    return pl.pallas_call(
        _final_body,
        out_shape=jax.ShapeDtypeStruct((G, C), jnp.float32),
        scratch_shapes=[
            pltpu.VMEM((NPAD, H), jnp.float32),
            pltpu.VMEM((NPAD, H), jnp.float32),
            pltpu.VMEM((NPAD, H), jnp.float32),
        ],
    )(p0, p1, hn, norm, W, b, batch_p, same_m, end_m, Wm1, Wm2, bm)


def kernel(x, edge_index, batch, W0, b0, W1, b1, W2, b2, Wm, bm):
    src = edge_index[0]
    dst = edge_index[1]
    pad_e = EPAD - E
    dst_p = jnp.concatenate(
        [dst, jnp.full((pad_e,), DUMMY, jnp.int32)]
    ).reshape(NCHUNKS, 128)
    src_p = jnp.concatenate(
        [src, jnp.zeros((pad_e,), jnp.int32)]
    ).reshape(NCHUNKS, 128)
    x_p = jnp.pad(x, ((0, NPAD - N), (0, 0)))
    batch_pad = jnp.pad(batch, (0, NPAD - N), constant_values=2**30)
    batch_p = batch_pad.reshape(1, NPAD)
    same_cols = [
        jnp.concatenate(
            [jnp.zeros((1 << k,), jnp.bool_), batch_pad[1 << k:] == batch_pad[:-(1 << k)]]
        )
        for k in range(NSCAN)
    ]
    same_m = jnp.stack(
        same_cols + [jnp.zeros((NPAD,), jnp.bool_)] * (16 - NSCAN), axis=1
    ).astype(jnp.float32)
    end_m = jnp.concatenate(
        [batch_pad[:-1] != batch_pad[1:], jnp.ones((1,), jnp.bool_)]
    ).astype(jnp.float32).reshape(1, NPAD)
    zeros_w = jnp.zeros((RPT, H), jnp.float32)
    ones_tab = jnp.ones((128, 128), jnp.float32)
    src_s = src_p.reshape(NSUB, SUB)
    dst_s = dst_p.reshape(NSUB, SUB)

    degp = _degree128(dst_p, ones_tab, zeros_w)
    norm, hn = _prep(degp[0], degp[1], x_p)
    p = _prop128(hn, src_s, dst_s, zeros_w)
    hn = _layer(p[0], p[1], hn, norm, W0, b0.reshape(1, H))
    p = _prop128(hn, src_s, dst_s, zeros_w)
    hn = _layer(p[0], p[1], hn, norm, W1, b1.reshape(1, H))
    p = _prop128(hn, src_s, dst_s, zeros_w)
    return _final(
        p[0], p[1], hn, norm, W2, b2.reshape(1, H),
        batch_p, same_m, end_m, Wm[:H], Wm[H:], bm.reshape(1, C),
    )
